# baseline (speedup 1.0000x reference)
import jax
import jax.numpy as jnp
from jax import lax
from jax.experimental import pallas as pl
from jax.experimental.pallas import tpu as pltpu
from jax.experimental.pallas import tpu_sc as plsc

N = 10000
NNZ = 320000
D = 128
DH = 64
LEAKY = 0.2
NC = 2
NS = 16
CHUNK = 128
SUPER = 2
STEP = CHUNK * SUPER
NCHUNKS = NNZ // STEP
CHUNKS_LO = NCHUNKS // NS
CHUNKS_REM = NCHUNKS % NS
GROUP = 40
NGROUPS = N // GROUP
GROUPS_LO = NGROUPS // NS
GROUPS_REM = NGROUPS % NS


def _body(p1_hbm, p2_hbm, emb_hbm, out_hbm, tmpst_hbm,
          acc_sp, pbuf, rows_v, sc_rows, obuf, gsem, ssem, msem):
    c = lax.axis_index("c")
    s = lax.axis_index("s")

    def _zero_row(r, carry):
        for k in range(DH // 16):
            obuf[r, pl.ds(k * 16, 16)] = jnp.zeros((16,), jnp.float32)
        return carry
    lax.fori_loop(0, GROUP, _zero_row, 0)
    n_groups = GROUPS_LO + jnp.where(s < GROUPS_REM, 1, 0)

    def _zero_group(i, carry):
        g0 = (i * NS + s) * GROUP
        pltpu.sync_copy(obuf, acc_sp.at[pl.ds(g0, GROUP)])
        return carry
    lax.fori_loop(0, n_groups, _zero_group, 0)
    plsc.subcore_barrier()

    base_chunk = s * CHUNKS_LO + jnp.minimum(s, CHUNKS_REM)
    n_chunks = CHUNKS_LO + jnp.where(s < CHUNKS_REM, 1, 0)

    def _scale_rows(ib4, ib2):
        for h in range(SUPER):
            def _group(j, carry):
                bits = pbuf[ib4, 2 * SUPER + h, pl.ds(j * 16, 16)]
                v16 = plsc.bitcast(bits, jnp.float32)
                for lane in range(16):
                    e = h * CHUNK + j * 16 + lane
                    sv = v16[lane]
                    for k in range(DH // 16):
                        sl = rows_v[ib2, e, pl.ds(k * 16, 16)]
                        sc_rows[ib2, e, pl.ds(k * 16, 16)] = sl * sv
                return carry
            lax.fori_loop(0, CHUNK // 16, _group, 0)

    def _run_phase(get_meta, gather_src, scatter_dst):
        def _meta(j):
            return pltpu.make_async_copy(
                get_meta(j), pbuf.at[lax.rem(j, 4)], msem.at[lax.rem(j, 2)])

        def _gather(j):
            jb = lax.rem(j, 4)
            j2 = lax.rem(j, 2)
            return [pltpu.make_async_copy(
                        gather_src.at[pbuf.at[jb, h]],
                        rows_v.at[j2, pl.ds(h * CHUNK, CHUNK)],
                        gsem.at[j2])
                    for h in range(SUPER)]

        def _scatter_start(j):
            jb = lax.rem(j, 4)
            j2 = lax.rem(j, 2)
            for h in range(SUPER):
                pltpu.async_copy(sc_rows.at[j2, pl.ds(h * CHUNK, CHUNK)],
                                 scatter_dst.at[pbuf.at[jb, SUPER + h]],
                                 ssem.at[j2], add=True)

        def _scatter_wait(j):
            jb = lax.rem(j, 4)
            j2 = lax.rem(j, 2)
            for h in range(SUPER):
                pltpu.make_async_copy(
                    sc_rows.at[j2, pl.ds(h * CHUNK, CHUNK)],
                    scatter_dst.at[pbuf.at[jb, SUPER + h]], ssem.at[j2]
                ).wait()

        _meta(0).start()
        _meta(1).start()
        _meta(0).wait()
        for d in _gather(0):
            d.start()

        def _step(i, carry):
            @pl.when(i >= 2)
            def _():
                _scatter_wait(i - 2)
            @pl.when(i + 2 < n_chunks)
            def _():
                _meta(i + 2).start()
            @pl.when(i + 1 < n_chunks)
            def _():
                _meta(i + 1).wait()
                for d in _gather(i + 1):
                    d.start()
            for d in _gather(i):
                d.wait()
            _scale_rows(lax.rem(i, 4), lax.rem(i, 2))
            _scatter_start(i)
            return carry
        lax.fori_loop(0, n_chunks, _step, 0)
        _scatter_wait(n_chunks - 2)
        _scatter_wait(n_chunks - 1)

    _run_phase(lambda i: p1_hbm.at[c, base_chunk + i], emb_hbm, acc_sp)
    plsc.subcore_barrier()

    def _stage_group(i, carry):
        g0 = (i * NS + s) * GROUP
        pltpu.sync_copy(acc_sp.at[pl.ds(g0, GROUP)],
                        tmpst_hbm.at[pl.ds(c * N + g0, GROUP)])
        pltpu.sync_copy(obuf, acc_sp.at[pl.ds(g0, GROUP)])
        return carry
    lax.fori_loop(0, n_groups, _stage_group, 0)
    plsc.subcore_barrier()

    _run_phase(lambda i: p2_hbm.at[c, base_chunk + i], tmpst_hbm, acc_sp)
    plsc.subcore_barrier()

    def _act_group(i, carry):
        g0 = (i * NS + s) * GROUP
        pltpu.sync_copy(acc_sp.at[pl.ds(g0, GROUP)], obuf)
        def _act_row(r, inner):
            for k in range(DH // 16):
                x = obuf[r, pl.ds(k * 16, 16)]
                obuf[r, pl.ds(k * 16, 16)] = jnp.maximum(x, x * LEAKY)
            return inner
        lax.fori_loop(0, GROUP, _act_row, 0)
        pltpu.sync_copy(obuf, out_hbm.at[c, pl.ds(g0, GROUP)])
        return carry
    lax.fori_loop(0, n_groups, _act_group, 0)


_sc_call = pl.kernel(
    _body,
    out_type=(jax.ShapeDtypeStruct((NC, N, DH), jnp.float32),
              jax.ShapeDtypeStruct((NC * N, DH), jnp.float32)),
    mesh=plsc.VectorSubcoreMesh(core_axis_name="c", subcore_axis_name="s"),
    compiler_params=pltpu.CompilerParams(use_tc_tiling_on_sc=False,
                                         needs_layout_passes=False),
    scratch_types=[
        pltpu.VMEM_SHARED((N, DH), jnp.float32),
        pltpu.VMEM((4, 3 * SUPER, CHUNK), jnp.int32),
        pltpu.VMEM((2, STEP, DH), jnp.float32),
        pltpu.VMEM((2, STEP, DH), jnp.float32),
        pltpu.VMEM((GROUP, DH), jnp.float32),
        pltpu.SemaphoreType.DMA((2,)),
        pltpu.SemaphoreType.DMA((2,)),
        pltpu.SemaphoreType.DMA((2,)),
    ],
)


@jax.jit
def kernel(adj_indices, adj_values, embs):
    row = adj_indices[0].astype(jnp.int32)
    col = adj_indices[1].astype(jnp.int32)
    emb2 = jnp.concatenate([embs[:, :DH], embs[:, DH:]], axis=0)
    colr = col.reshape(NCHUNKS, SUPER, CHUNK)
    rowr = row.reshape(NCHUNKS, SUPER, CHUNK)
    bits = lax.bitcast_convert_type(adj_values, jnp.int32).reshape(
        NCHUNKS, SUPER, CHUNK)
    p1 = jnp.stack([
        jnp.concatenate([rowr + cc * N, colr, bits], axis=1)
        for cc in range(NC)
    ])
    p2 = jnp.stack([
        jnp.concatenate([colr + cc * N, rowr, bits], axis=1)
        for cc in range(NC)
    ])
    out2, _ = _sc_call(p1, p2, emb2)
    return jnp.concatenate([out2[0], out2[1]], axis=1)


# speedup vs baseline: 1.8999x; 1.8999x over previous
import jax
import jax.numpy as jnp
from jax import lax
from jax.experimental import pallas as pl
from jax.experimental.pallas import tpu as pltpu
from jax.experimental.pallas import tpu_sc as plsc

N = 10000
NNZ = 320000
D = 128
DH = 64
LEAKY = 0.2
NC = 2
NS = 16
CHUNK = 128
SUPER = 2
STEP = CHUNK * SUPER
NCHUNKS = NNZ // STEP
CHUNKS_LO = NCHUNKS // NS
CHUNKS_REM = NCHUNKS % NS
GROUP = 40
NGROUPS = N // GROUP
GROUPS_LO = NGROUPS // NS
GROUPS_REM = NGROUPS % NS


def _body(p1_hbm, p2_hbm, emb_hbm, out_hbm, tmpst_hbm,
          acc_sp, pbuf, rows_v, obuf, gsem, ssem, msem):
    c = lax.axis_index("c")
    s = lax.axis_index("s")

    def _zero_row(r, carry):
        for k in range(DH // 16):
            obuf[r, pl.ds(k * 16, 16)] = jnp.zeros((16,), jnp.float32)
        return carry
    lax.fori_loop(0, GROUP, _zero_row, 0)
    n_groups = GROUPS_LO + jnp.where(s < GROUPS_REM, 1, 0)

    def _zero_group(i, carry):
        g0 = (i * NS + s) * GROUP
        pltpu.sync_copy(obuf, acc_sp.at[pl.ds(g0, GROUP)])
        return carry
    lax.fori_loop(0, n_groups, _zero_group, 0)
    plsc.subcore_barrier()

    base_chunk = s * CHUNKS_LO + jnp.minimum(s, CHUNKS_REM)
    n_chunks = CHUNKS_LO + jnp.where(s < CHUNKS_REM, 1, 0)

    def _scale_rows(ib):
        for h in range(SUPER):
            @plsc.parallel_loop(0, CHUNK // 16, 1, unroll=2)
            def _group(j):
                bits = pbuf[ib, 2 * SUPER + h, pl.ds(j * 16, 16)]
                v16 = plsc.bitcast(bits, jnp.float32)
                for lane in range(16):
                    e = h * CHUNK + j * 16 + lane
                    sv = v16[lane]
                    for k in range(DH // 16):
                        sl = rows_v[ib, e, pl.ds(k * 16, 16)]
                        rows_v[ib, e, pl.ds(k * 16, 16)] = sl * sv

    def _run_phase(get_meta, gather_src, scatter_dst):
        def _meta(j):
            return pltpu.make_async_copy(
                get_meta(j), pbuf.at[lax.rem(j, 4)], msem.at[lax.rem(j, 2)])

        def _gather(j):
            jb = lax.rem(j, 4)
            return [pltpu.make_async_copy(
                        gather_src.at[pbuf.at[jb, h]],
                        rows_v.at[jb, pl.ds(h * CHUNK, CHUNK)],
                        gsem.at[jb])
                    for h in range(SUPER)]

        def _scatter_start(j):
            jb = lax.rem(j, 4)
            for h in range(SUPER):
                pltpu.async_copy(rows_v.at[jb, pl.ds(h * CHUNK, CHUNK)],
                                 scatter_dst.at[pbuf.at[jb, SUPER + h]],
                                 ssem.at[jb], add=True)

        def _scatter_wait(j):
            jb = lax.rem(j, 4)
            for h in range(SUPER):
                pltpu.make_async_copy(
                    rows_v.at[jb, pl.ds(h * CHUNK, CHUNK)],
                    scatter_dst.at[pbuf.at[jb, SUPER + h]], ssem.at[jb]
                ).wait()

        _meta(0).start()
        _meta(1).start()
        _meta(0).wait()
        for d in _gather(0):
            d.start()

        def _step(i, carry):
            @pl.when(i >= 2)
            def _():
                _scatter_wait(i - 2)
            @pl.when(i + 2 < n_chunks)
            def _():
                _meta(i + 2).start()
            @pl.when(i + 1 < n_chunks)
            def _():
                _meta(i + 1).wait()
                for d in _gather(i + 1):
                    d.start()
            for d in _gather(i):
                d.wait()
            _scale_rows(lax.rem(i, 4))
            _scatter_start(i)
            return carry
        lax.fori_loop(0, n_chunks, _step, 0)
        _scatter_wait(n_chunks - 2)
        _scatter_wait(n_chunks - 1)

    _run_phase(lambda i: p1_hbm.at[c, base_chunk + i], emb_hbm, acc_sp)
    plsc.subcore_barrier()

    def _stage_group(i, carry):
        g0 = (i * NS + s) * GROUP
        pltpu.sync_copy(acc_sp.at[pl.ds(g0, GROUP)],
                        tmpst_hbm.at[pl.ds(c * N + g0, GROUP)])
        pltpu.sync_copy(obuf, acc_sp.at[pl.ds(g0, GROUP)])
        return carry
    lax.fori_loop(0, n_groups, _stage_group, 0)
    plsc.subcore_barrier()

    _run_phase(lambda i: p2_hbm.at[c, base_chunk + i], tmpst_hbm, acc_sp)
    plsc.subcore_barrier()

    def _act_group(i, carry):
        g0 = (i * NS + s) * GROUP
        pltpu.sync_copy(acc_sp.at[pl.ds(g0, GROUP)], obuf)
        def _act_row(r, inner):
            for k in range(DH // 16):
                x = obuf[r, pl.ds(k * 16, 16)]
                obuf[r, pl.ds(k * 16, 16)] = jnp.maximum(x, x * LEAKY)
            return inner
        lax.fori_loop(0, GROUP, _act_row, 0)
        pltpu.sync_copy(obuf, out_hbm.at[c, pl.ds(g0, GROUP)])
        return carry
    lax.fori_loop(0, n_groups, _act_group, 0)


_sc_call = pl.kernel(
    _body,
    out_type=(jax.ShapeDtypeStruct((NC, N, DH), jnp.float32),
              jax.ShapeDtypeStruct((NC * N, DH), jnp.float32)),
    mesh=plsc.VectorSubcoreMesh(core_axis_name="c", subcore_axis_name="s"),
    compiler_params=pltpu.CompilerParams(use_tc_tiling_on_sc=False,
                                         needs_layout_passes=False),
    scratch_types=[
        pltpu.VMEM_SHARED((N, DH), jnp.float32),
        pltpu.VMEM((4, 3 * SUPER, CHUNK), jnp.int32),
        pltpu.VMEM((4, STEP, DH), jnp.float32),
        pltpu.VMEM((GROUP, DH), jnp.float32),
        pltpu.SemaphoreType.DMA((4,)),
        pltpu.SemaphoreType.DMA((4,)),
        pltpu.SemaphoreType.DMA((2,)),
    ],
)


@jax.jit
def kernel(adj_indices, adj_values, embs):
    row = adj_indices[0].astype(jnp.int32)
    col = adj_indices[1].astype(jnp.int32)
    emb2 = jnp.concatenate([embs[:, :DH], embs[:, DH:]], axis=0)
    colr = col.reshape(NCHUNKS, SUPER, CHUNK)
    rowr = row.reshape(NCHUNKS, SUPER, CHUNK)
    bits = lax.bitcast_convert_type(adj_values, jnp.int32).reshape(
        NCHUNKS, SUPER, CHUNK)
    p1 = jnp.stack([
        jnp.concatenate([rowr + cc * N, colr, bits], axis=1)
        for cc in range(NC)
    ])
    p2 = jnp.stack([
        jnp.concatenate([colr + cc * N, rowr, bits], axis=1)
        for cc in range(NC)
    ])
    out2, _ = _sc_call(p1, p2, emb2)
    return jnp.concatenate([out2[0], out2[1]], axis=1)


# speedup vs baseline: 2.3757x; 1.2505x over previous
import jax
import jax.numpy as jnp
from jax import lax
from jax.experimental import pallas as pl
from jax.experimental.pallas import tpu as pltpu
from jax.experimental.pallas import tpu_sc as plsc

N = 10000
NNZ = 320000
D = 128
DH = 64
LEAKY = 0.2
NC = 2
NS = 16
CHUNK = 128
SUPER = 2
STEP = CHUNK * SUPER
NCHUNKS = NNZ // STEP
CHUNKS_LO = NCHUNKS // NS
CHUNKS_REM = NCHUNKS % NS
GROUP = 40
NGROUPS = N // GROUP
GROUPS_LO = NGROUPS // NS
GROUPS_REM = NGROUPS % NS


def _body(p1_hbm, p2_hbm, emb_hbm, out_hbm, tmpst_hbm,
          acc_sp, pbuf, rows_v, obuf, gsem, ssem, msem):
    c = lax.axis_index("c")
    s = lax.axis_index("s")

    def _zero_row(r, carry):
        for k in range(DH // 16):
            obuf[r, pl.ds(k * 16, 16)] = jnp.zeros((16,), jnp.float32)
        return carry
    lax.fori_loop(0, GROUP, _zero_row, 0)
    n_groups = GROUPS_LO + jnp.where(s < GROUPS_REM, 1, 0)

    def _zero_group(i, carry):
        g0 = (i * NS + s) * GROUP
        pltpu.sync_copy(obuf, acc_sp.at[pl.ds(g0, GROUP)])
        return carry
    lax.fori_loop(0, n_groups, _zero_group, 0)
    plsc.subcore_barrier()

    base_chunk = s * CHUNKS_LO + jnp.minimum(s, CHUNKS_REM)
    n_chunks = CHUNKS_LO + jnp.where(s < CHUNKS_REM, 1, 0)

    def _scale_rows(ib):
        for h in range(SUPER):
            @plsc.parallel_loop(0, CHUNK // 16, 1, unroll=4)
            def _group(j):
                bits = pbuf[ib, 2 * SUPER + h, pl.ds(j * 16, 16)]
                v16 = plsc.bitcast(bits, jnp.float32)
                for lane in range(16):
                    e = h * CHUNK + j * 16 + lane
                    sv = v16[lane]
                    for k in range(DH // 16):
                        sl = rows_v[ib, e, pl.ds(k * 16, 16)]
                        rows_v[ib, e, pl.ds(k * 16, 16)] = sl * sv

    def _run_phase(get_meta, gather_src, scatter_dst):
        def _meta(j):
            return pltpu.make_async_copy(
                get_meta(j), pbuf.at[lax.rem(j, 4)], msem.at[lax.rem(j, 2)])

        def _gather(j):
            jb = lax.rem(j, 4)
            return [pltpu.make_async_copy(
                        gather_src.at[pbuf.at[jb, h]],
                        rows_v.at[jb, pl.ds(h * CHUNK, CHUNK)],
                        gsem.at[jb])
                    for h in range(SUPER)]

        def _scatter_start(j):
            jb = lax.rem(j, 4)
            for h in range(SUPER):
                pltpu.async_copy(rows_v.at[jb, pl.ds(h * CHUNK, CHUNK)],
                                 scatter_dst.at[pbuf.at[jb, SUPER + h]],
                                 ssem.at[jb], add=True)

        def _scatter_wait(j):
            jb = lax.rem(j, 4)
            for h in range(SUPER):
                pltpu.make_async_copy(
                    rows_v.at[jb, pl.ds(h * CHUNK, CHUNK)],
                    scatter_dst.at[pbuf.at[jb, SUPER + h]], ssem.at[jb]
                ).wait()

        _meta(0).start()
        _meta(1).start()
        _meta(0).wait()
        for d in _gather(0):
            d.start()

        def _step(i, carry):
            @pl.when(i >= 2)
            def _():
                _scatter_wait(i - 2)
            @pl.when(i + 2 < n_chunks)
            def _():
                _meta(i + 2).start()
            @pl.when(i + 1 < n_chunks)
            def _():
                _meta(i + 1).wait()
                for d in _gather(i + 1):
                    d.start()
            for d in _gather(i):
                d.wait()
            _scale_rows(lax.rem(i, 4))
            _scatter_start(i)
            return carry
        lax.fori_loop(0, n_chunks, _step, 0)
        _scatter_wait(n_chunks - 2)
        _scatter_wait(n_chunks - 1)

    _run_phase(lambda i: p1_hbm.at[c, base_chunk + i], emb_hbm, acc_sp)
    plsc.subcore_barrier()

    def _stage_group(i, carry):
        g0 = (i * NS + s) * GROUP
        pltpu.sync_copy(acc_sp.at[pl.ds(g0, GROUP)],
                        tmpst_hbm.at[pl.ds(c * N + g0, GROUP)])
        pltpu.sync_copy(obuf, acc_sp.at[pl.ds(g0, GROUP)])
        return carry
    lax.fori_loop(0, n_groups, _stage_group, 0)
    plsc.subcore_barrier()

    _run_phase(lambda i: p2_hbm.at[c, base_chunk + i], tmpst_hbm, acc_sp)
    plsc.subcore_barrier()

    def _act_group(i, carry):
        g0 = (i * NS + s) * GROUP
        pltpu.sync_copy(acc_sp.at[pl.ds(g0, GROUP)], obuf)
        def _act_row(r, inner):
            for k in range(DH // 16):
                x = obuf[r, pl.ds(k * 16, 16)]
                obuf[r, pl.ds(k * 16, 16)] = jnp.maximum(x, x * LEAKY)
            return inner
        lax.fori_loop(0, GROUP, _act_row, 0)
        pltpu.sync_copy(obuf, out_hbm.at[c, pl.ds(g0, GROUP)])
        return carry
    lax.fori_loop(0, n_groups, _act_group, 0)


_sc_call = pl.kernel(
    _body,
    out_type=(jax.ShapeDtypeStruct((NC, N, DH), jnp.float32),
              jax.ShapeDtypeStruct((NC * N, DH), jnp.float32)),
    mesh=plsc.VectorSubcoreMesh(core_axis_name="c", subcore_axis_name="s"),
    compiler_params=pltpu.CompilerParams(use_tc_tiling_on_sc=False,
                                         needs_layout_passes=False),
    scratch_types=[
        pltpu.VMEM_SHARED((N, DH), jnp.float32),
        pltpu.VMEM((4, 3 * SUPER, CHUNK), jnp.int32),
        pltpu.VMEM((4, STEP, DH), jnp.float32),
        pltpu.VMEM((GROUP, DH), jnp.float32),
        pltpu.SemaphoreType.DMA((4,)),
        pltpu.SemaphoreType.DMA((4,)),
        pltpu.SemaphoreType.DMA((2,)),
    ],
)


@jax.jit
def kernel(adj_indices, adj_values, embs):
    row = adj_indices[0].astype(jnp.int32)
    col = adj_indices[1].astype(jnp.int32)
    emb2 = jnp.concatenate([embs[:, :DH], embs[:, DH:]], axis=0)
    colr = col.reshape(NCHUNKS, SUPER, CHUNK)
    rowr = row.reshape(NCHUNKS, SUPER, CHUNK)
    bits = lax.bitcast_convert_type(adj_values, jnp.int32).reshape(
        NCHUNKS, SUPER, CHUNK)
    p1 = jnp.stack([
        jnp.concatenate([rowr + cc * N, colr, bits], axis=1)
        for cc in range(NC)
    ])
    p2 = jnp.stack([
        jnp.concatenate([colr + cc * N, rowr, bits], axis=1)
        for cc in range(NC)
    ])
    out2, _ = _sc_call(p1, p2, emb2)
    return jnp.concatenate([out2[0], out2[1]], axis=1)


# speedup vs baseline: 2.6070x; 1.0974x over previous
import jax
import jax.numpy as jnp
from jax import lax
from jax.experimental import pallas as pl
from jax.experimental.pallas import tpu as pltpu
from jax.experimental.pallas import tpu_sc as plsc

N = 10000
NNZ = 320000
D = 128
DH = 64
LEAKY = 0.2
NC = 2
NS = 16
CHUNK = 128
SUPER = 2
STEP = CHUNK * SUPER
NCHUNKS = NNZ // STEP
CHUNKS_LO = NCHUNKS // NS
CHUNKS_REM = NCHUNKS % NS
GROUP = 40
NGROUPS = N // GROUP
GROUPS_LO = NGROUPS // NS
GROUPS_REM = NGROUPS % NS


def _body(p1_hbm, p2_hbm, emb_hbm, out_hbm, tmpst_hbm,
          acc_sp, pbuf, rows_v, obuf, gsem, ssem, msem):
    c = lax.axis_index("c")
    s = lax.axis_index("s")

    def _zero_row(r, carry):
        for k in range(DH // 16):
            obuf[r, pl.ds(k * 16, 16)] = jnp.zeros((16,), jnp.float32)
        return carry
    lax.fori_loop(0, GROUP, _zero_row, 0)
    n_groups = GROUPS_LO + jnp.where(s < GROUPS_REM, 1, 0)

    def _zero_group(i, carry):
        g0 = (i * NS + s) * GROUP
        pltpu.sync_copy(obuf, acc_sp.at[pl.ds(g0, GROUP)])
        return carry
    lax.fori_loop(0, n_groups, _zero_group, 0)
    plsc.subcore_barrier()

    base_chunk = s * CHUNKS_LO + jnp.minimum(s, CHUNKS_REM)
    n_chunks = CHUNKS_LO + jnp.where(s < CHUNKS_REM, 1, 0)

    def _scale_rows(ib):
        for h in range(SUPER):
            @plsc.parallel_loop(0, CHUNK // 16, 1, unroll=8)
            def _group(j):
                bits = pbuf[ib, 2 * SUPER + h, pl.ds(j * 16, 16)]
                v16 = plsc.bitcast(bits, jnp.float32)
                for lane in range(16):
                    e = h * CHUNK + j * 16 + lane
                    sv = v16[lane]
                    for k in range(DH // 16):
                        sl = rows_v[ib, e, pl.ds(k * 16, 16)]
                        rows_v[ib, e, pl.ds(k * 16, 16)] = sl * sv

    def _run_phase(get_meta, gather_src, scatter_dst):
        def _meta(j):
            return pltpu.make_async_copy(
                get_meta(j), pbuf.at[lax.rem(j, 4)], msem.at[lax.rem(j, 2)])

        def _gather(j):
            jb = lax.rem(j, 4)
            return [pltpu.make_async_copy(
                        gather_src.at[pbuf.at[jb, h]],
                        rows_v.at[jb, pl.ds(h * CHUNK, CHUNK)],
                        gsem.at[jb])
                    for h in range(SUPER)]

        def _scatter_start(j):
            jb = lax.rem(j, 4)
            for h in range(SUPER):
                pltpu.async_copy(rows_v.at[jb, pl.ds(h * CHUNK, CHUNK)],
                                 scatter_dst.at[pbuf.at[jb, SUPER + h]],
                                 ssem.at[jb], add=True)

        def _scatter_wait(j):
            jb = lax.rem(j, 4)
            for h in range(SUPER):
                pltpu.make_async_copy(
                    rows_v.at[jb, pl.ds(h * CHUNK, CHUNK)],
                    scatter_dst.at[pbuf.at[jb, SUPER + h]], ssem.at[jb]
                ).wait()

        _meta(0).start()
        _meta(1).start()
        _meta(0).wait()
        for d in _gather(0):
            d.start()

        def _step(i, carry):
            @pl.when(i >= 2)
            def _():
                _scatter_wait(i - 2)
            @pl.when(i + 2 < n_chunks)
            def _():
                _meta(i + 2).start()
            @pl.when(i + 1 < n_chunks)
            def _():
                _meta(i + 1).wait()
                for d in _gather(i + 1):
                    d.start()
            for d in _gather(i):
                d.wait()
            _scale_rows(lax.rem(i, 4))
            _scatter_start(i)
            return carry
        lax.fori_loop(0, n_chunks, _step, 0)
        _scatter_wait(n_chunks - 2)
        _scatter_wait(n_chunks - 1)

    _run_phase(lambda i: p1_hbm.at[c, base_chunk + i], emb_hbm, acc_sp)
    plsc.subcore_barrier()

    def _stage_group(i, carry):
        g0 = (i * NS + s) * GROUP
        pltpu.sync_copy(acc_sp.at[pl.ds(g0, GROUP)],
                        tmpst_hbm.at[pl.ds(c * N + g0, GROUP)])
        pltpu.sync_copy(obuf, acc_sp.at[pl.ds(g0, GROUP)])
        return carry
    lax.fori_loop(0, n_groups, _stage_group, 0)
    plsc.subcore_barrier()

    _run_phase(lambda i: p2_hbm.at[c, base_chunk + i], tmpst_hbm, acc_sp)
    plsc.subcore_barrier()

    def _act_group(i, carry):
        g0 = (i * NS + s) * GROUP
        pltpu.sync_copy(acc_sp.at[pl.ds(g0, GROUP)], obuf)
        def _act_row(r, inner):
            for k in range(DH // 16):
                x = obuf[r, pl.ds(k * 16, 16)]
                obuf[r, pl.ds(k * 16, 16)] = jnp.maximum(x, x * LEAKY)
            return inner
        lax.fori_loop(0, GROUP, _act_row, 0)
        pltpu.sync_copy(obuf, out_hbm.at[c, pl.ds(g0, GROUP)])
        return carry
    lax.fori_loop(0, n_groups, _act_group, 0)


_sc_call = pl.kernel(
    _body,
    out_type=(jax.ShapeDtypeStruct((NC, N, DH), jnp.float32),
              jax.ShapeDtypeStruct((NC * N, DH), jnp.float32)),
    mesh=plsc.VectorSubcoreMesh(core_axis_name="c", subcore_axis_name="s"),
    compiler_params=pltpu.CompilerParams(use_tc_tiling_on_sc=False,
                                         needs_layout_passes=False),
    scratch_types=[
        pltpu.VMEM_SHARED((N, DH), jnp.float32),
        pltpu.VMEM((4, 3 * SUPER, CHUNK), jnp.int32),
        pltpu.VMEM((4, STEP, DH), jnp.float32),
        pltpu.VMEM((GROUP, DH), jnp.float32),
        pltpu.SemaphoreType.DMA((4,)),
        pltpu.SemaphoreType.DMA((4,)),
        pltpu.SemaphoreType.DMA((2,)),
    ],
)


@jax.jit
def kernel(adj_indices, adj_values, embs):
    row = adj_indices[0].astype(jnp.int32)
    col = adj_indices[1].astype(jnp.int32)
    emb2 = jnp.concatenate([embs[:, :DH], embs[:, DH:]], axis=0)
    colr = col.reshape(NCHUNKS, SUPER, CHUNK)
    rowr = row.reshape(NCHUNKS, SUPER, CHUNK)
    bits = lax.bitcast_convert_type(adj_values, jnp.int32).reshape(
        NCHUNKS, SUPER, CHUNK)
    p1 = jnp.stack([
        jnp.concatenate([rowr + cc * N, colr, bits], axis=1)
        for cc in range(NC)
    ])
    p2 = jnp.stack([
        jnp.concatenate([colr + cc * N, rowr, bits], axis=1)
        for cc in range(NC)
    ])
    out2, _ = _sc_call(p1, p2, emb2)
    return jnp.concatenate([out2[0], out2[1]], axis=1)
